# Initial kernel scaffold; baseline (speedup 1.0000x reference)
import jax
import jax.numpy as jnp
from jax.experimental import pallas as pl


def kernel(embedding_logits, semantic_labels, instance_labels, feature_dim):
    raise NotImplementedError("write your pallas kernel here")


# speedup vs baseline: 7.7351x; 7.7351x over previous
import functools

import jax
import jax.numpy as jnp
from jax.experimental import pallas as pl
from jax.experimental.pallas import tpu as pltpu

_IGNORE = 0
_DELTA_V = 0.5
_DELTA_D = 1.5
_PARAM_VAR = 1.0
_PARAM_DIST = 1.0
_PARAM_REG = 0.001
_LOSS_WEIGHT = 1.0
_L = 50

_C = 2048


def _dl_kernel(pred_ref, sem_ref, inst_ref, out_ref,
               seg_ref, cnt_ref, mu_ref, lv_ref, *, n, klast):
    p = pl.program_id(1)
    k = pl.program_id(2)
    f = pred_ref.shape[1]

    x = pred_ref[0]
    sem = sem_ref[0]
    inst = inst_ref[0]

    col = jax.lax.broadcasted_iota(jnp.int32, (1, _C), 1)
    valid = (k * _C + col) < n
    keep = jnp.logical_and(sem != _IGNORE, valid)
    w = keep.astype(jnp.float32)
    label = jnp.where(sem == 1, 0, inst)
    label = jnp.where(keep, label, 0)

    x = jnp.where(valid, x, 0.0)

    rows = jax.lax.broadcasted_iota(jnp.int32, (_L, _C), 0)
    onehot_w = jnp.where(label == rows, w, 0.0)

    @pl.when(jnp.logical_and(p == 0, k == 0))
    def _init_pass0():
        seg_ref[...] = jnp.zeros_like(seg_ref)
        cnt_ref[...] = jnp.zeros_like(cnt_ref)

    @pl.when(p == 0)
    def _pass0():
        seg_ref[...] += jax.lax.dot_general(
            onehot_w, x, (((1,), (1,)), ((), ())),
            preferred_element_type=jnp.float32,
            precision=jax.lax.Precision.HIGHEST)
        ones = jnp.ones((1, _C), jnp.float32)
        cnt_ref[...] += jax.lax.dot_general(
            onehot_w, ones, (((1,), (1,)), ((), ())),
            preferred_element_type=jnp.float32,
            precision=jax.lax.Precision.HIGHEST)

    @pl.when(jnp.logical_and(p == 1, k == 0))
    def _init_pass1():
        cnt = cnt_ref[...]
        mu_ref[...] = seg_ref[...] / (cnt + 1e-8)
        lv_ref[...] = jnp.zeros_like(lv_ref)

    @pl.when(p == 1)
    def _pass1():
        mu = mu_ref[...]
        mu_t = mu.T
        mu_exp = jax.lax.dot_general(
            mu_t, onehot_w, (((1,), (0,)), ((), ())),
            preferred_element_type=jnp.float32,
            precision=jax.lax.Precision.HIGHEST)
        dist = jnp.sum(jnp.abs(x - mu_exp), axis=0, keepdims=True)
        dist = jnp.square(jnp.maximum(dist - _DELTA_V, 0.0)) * w
        lv_ref[...] += jax.lax.dot_general(
            onehot_w, dist, (((1,), (1,)), ((), ())),
            preferred_element_type=jnp.float32,
            precision=jax.lax.Precision.HIGHEST)

    @pl.when(jnp.logical_and(p == 1, k == klast))
    def _finish():
        cnt = cnt_ref[...]
        mu = mu_ref[...]
        present = (cnt > 0.0).astype(jnp.float32)
        ninst = jnp.sum(present)
        l_var = jnp.sum(present * lv_ref[...] / (cnt + 1e-8)) / ninst

        mu_t = mu.T
        norm = jnp.zeros((_L, _L), jnp.float32)
        for j in range(f):
            norm = norm + jnp.abs(mu[:, j:j + 1] - mu_t[j:j + 1, :])
        hinge = jnp.square(jnp.maximum(2.0 * _DELTA_D - norm, 0.0))
        ii = jax.lax.broadcasted_iota(jnp.int32, (_L, _L), 0)
        jj = jax.lax.broadcasted_iota(jnp.int32, (_L, _L), 1)
        pair_mask = present * present.T * jnp.where(ii == jj, 0.0, 1.0)
        l_dist = jnp.sum(pair_mask * hinge) / jnp.sum(pair_mask)

        l_reg = jnp.sum(present * jnp.sum(jnp.abs(mu), axis=1, keepdims=True)) / ninst

        r = jax.lax.broadcasted_iota(jnp.int32, (8, 128), 0)
        c = jax.lax.broadcasted_iota(jnp.int32, (8, 128), 1)
        first = (c == 0)
        packed = (jnp.where(jnp.logical_and(r == 0, first), l_var, 0.0)
                  + jnp.where(jnp.logical_and(r == 1, first), l_dist, 0.0)
                  + jnp.where(jnp.logical_and(r == 2, first), l_reg, 0.0))
        out_ref[0] = packed


def kernel(embedding_logits, semantic_labels, instance_labels, feature_dim):
    b, f, n = embedding_logits.shape
    k = -(-n // _C)
    sem3 = semantic_labels.reshape(b, 1, n)
    inst3 = instance_labels.reshape(b, 1, n)
    out = pl.pallas_call(
        functools.partial(_dl_kernel, n=n, klast=k - 1),
        grid=(b, 2, k),
        in_specs=[
            pl.BlockSpec((1, f, _C), lambda bi, pi, ki: (bi, 0, ki)),
            pl.BlockSpec((1, 1, _C), lambda bi, pi, ki: (bi, 0, ki)),
            pl.BlockSpec((1, 1, _C), lambda bi, pi, ki: (bi, 0, ki)),
        ],
        out_specs=pl.BlockSpec((1, 8, 128), lambda bi, pi, ki: (bi, 0, 0)),
        out_shape=jax.ShapeDtypeStruct((b, 8, 128), jnp.float32),
        scratch_shapes=[
            pltpu.VMEM((_L, f), jnp.float32),
            pltpu.VMEM((_L, 1), jnp.float32),
            pltpu.VMEM((_L, f), jnp.float32),
            pltpu.VMEM((_L, 1), jnp.float32),
        ],
    )(embedding_logits, sem3, inst3)
    l_var = _PARAM_VAR * out[:, 0, 0]
    l_dist = _PARAM_DIST * out[:, 1, 0]
    l_reg = _PARAM_REG * out[:, 2, 0]
    loss = _LOSS_WEIGHT * (l_var + l_dist + l_reg)
    scale = (jnp.asarray(feature_dim) // f).astype(jnp.float32)
    return (jnp.mean(loss) * scale, jnp.mean(l_var) * scale,
            jnp.mean(l_dist) * scale, jnp.mean(l_reg) * scale)


# speedup vs baseline: 12.3793x; 1.6004x over previous
import functools

import jax
import jax.numpy as jnp
from jax.experimental import pallas as pl
from jax.experimental.pallas import tpu as pltpu

_IGNORE = 0
_DELTA_V = 0.5
_DELTA_D = 1.5
_PARAM_VAR = 1.0
_PARAM_DIST = 1.0
_PARAM_REG = 0.001
_LOSS_WEIGHT = 1.0
_L = 50

_C = 8192


def _dl_kernel(pred_ref, sem_ref, inst_ref, out_ref,
               seg_ref, cnt_ref, mu_ref, lv_ref, *, n, klast):
    p = pl.program_id(1)
    k = pl.program_id(2)
    f = pred_ref.shape[1]

    x = pred_ref[0]
    sem = sem_ref[0]
    inst = inst_ref[0]

    col = jax.lax.broadcasted_iota(jnp.int32, (1, _C), 1)
    valid = (k * _C + col) < n
    keep = jnp.logical_and(sem != _IGNORE, valid)
    w = keep.astype(jnp.float32)
    label = jnp.where(sem == 1, 0, inst)
    label = jnp.where(keep, label, 0)

    x = jnp.where(valid, x, 0.0)

    rows = jax.lax.broadcasted_iota(jnp.int32, (_L, _C), 0)
    onehot_w = jnp.where(label == rows, w, 0.0)

    @pl.when(jnp.logical_and(p == 0, k == 0))
    def _init_pass0():
        seg_ref[...] = jnp.zeros_like(seg_ref)
        cnt_ref[...] = jnp.zeros_like(cnt_ref)

    @pl.when(p == 0)
    def _pass0():
        seg_ref[...] += jax.lax.dot_general(
            onehot_w, x, (((1,), (1,)), ((), ())),
            preferred_element_type=jnp.float32,
            precision=jax.lax.Precision.HIGHEST)
        ones = jnp.ones((1, _C), jnp.float32)
        cnt_ref[...] += jax.lax.dot_general(
            onehot_w, ones, (((1,), (1,)), ((), ())),
            preferred_element_type=jnp.float32,
            precision=jax.lax.Precision.HIGHEST)

    @pl.when(jnp.logical_and(p == 1, k == 0))
    def _init_pass1():
        cnt = cnt_ref[...]
        mu_ref[...] = seg_ref[...] / (cnt + 1e-8)
        lv_ref[...] = jnp.zeros_like(lv_ref)

    @pl.when(p == 1)
    def _pass1():
        mu = mu_ref[...]
        mu_t = mu.T
        mu_exp = jax.lax.dot_general(
            mu_t, onehot_w, (((1,), (0,)), ((), ())),
            preferred_element_type=jnp.float32,
            precision=jax.lax.Precision.HIGHEST)
        dist = jnp.sum(jnp.abs(x - mu_exp), axis=0, keepdims=True)
        dist = jnp.square(jnp.maximum(dist - _DELTA_V, 0.0)) * w
        lv_ref[...] += jax.lax.dot_general(
            onehot_w, dist, (((1,), (1,)), ((), ())),
            preferred_element_type=jnp.float32,
            precision=jax.lax.Precision.HIGHEST)

    @pl.when(jnp.logical_and(p == 1, k == klast))
    def _finish():
        cnt = cnt_ref[...]
        mu = mu_ref[...]
        present = (cnt > 0.0).astype(jnp.float32)
        ninst = jnp.sum(present)
        l_var = jnp.sum(present * lv_ref[...] / (cnt + 1e-8)) / ninst

        mu_t = mu.T
        norm = jnp.zeros((_L, _L), jnp.float32)
        for j in range(f):
            norm = norm + jnp.abs(mu[:, j:j + 1] - mu_t[j:j + 1, :])
        hinge = jnp.square(jnp.maximum(2.0 * _DELTA_D - norm, 0.0))
        ii = jax.lax.broadcasted_iota(jnp.int32, (_L, _L), 0)
        jj = jax.lax.broadcasted_iota(jnp.int32, (_L, _L), 1)
        pair_mask = present * present.T * jnp.where(ii == jj, 0.0, 1.0)
        l_dist = jnp.sum(pair_mask * hinge) / jnp.sum(pair_mask)

        l_reg = jnp.sum(present * jnp.sum(jnp.abs(mu), axis=1, keepdims=True)) / ninst

        r = jax.lax.broadcasted_iota(jnp.int32, (8, 128), 0)
        c = jax.lax.broadcasted_iota(jnp.int32, (8, 128), 1)
        first = (c == 0)
        packed = (jnp.where(jnp.logical_and(r == 0, first), l_var, 0.0)
                  + jnp.where(jnp.logical_and(r == 1, first), l_dist, 0.0)
                  + jnp.where(jnp.logical_and(r == 2, first), l_reg, 0.0))
        out_ref[0] = packed


def kernel(embedding_logits, semantic_labels, instance_labels, feature_dim):
    b, f, n = embedding_logits.shape
    k = -(-n // _C)
    sem3 = semantic_labels.reshape(b, 1, n)
    inst3 = instance_labels.reshape(b, 1, n)
    out = pl.pallas_call(
        functools.partial(_dl_kernel, n=n, klast=k - 1),
        grid=(b, 2, k),
        in_specs=[
            pl.BlockSpec((1, f, _C), lambda bi, pi, ki: (bi, 0, ki)),
            pl.BlockSpec((1, 1, _C), lambda bi, pi, ki: (bi, 0, ki)),
            pl.BlockSpec((1, 1, _C), lambda bi, pi, ki: (bi, 0, ki)),
        ],
        out_specs=pl.BlockSpec((1, 8, 128), lambda bi, pi, ki: (bi, 0, 0)),
        out_shape=jax.ShapeDtypeStruct((b, 8, 128), jnp.float32),
        scratch_shapes=[
            pltpu.VMEM((_L, f), jnp.float32),
            pltpu.VMEM((_L, 1), jnp.float32),
            pltpu.VMEM((_L, f), jnp.float32),
            pltpu.VMEM((_L, 1), jnp.float32),
        ],
    )(embedding_logits, sem3, inst3)
    l_var = _PARAM_VAR * out[:, 0, 0]
    l_dist = _PARAM_DIST * out[:, 1, 0]
    l_reg = _PARAM_REG * out[:, 2, 0]
    loss = _LOSS_WEIGHT * (l_var + l_dist + l_reg)
    scale = (jnp.asarray(feature_dim) // f).astype(jnp.float32)
    return (jnp.mean(loss) * scale, jnp.mean(l_var) * scale,
            jnp.mean(l_dist) * scale, jnp.mean(l_reg) * scale)


# speedup vs baseline: 18.0684x; 1.4596x over previous
import functools

import jax
import jax.numpy as jnp
from jax.experimental import pallas as pl
from jax.experimental.pallas import tpu as pltpu

_IGNORE = 0
_DELTA_V = 0.5
_DELTA_D = 1.5
_PARAM_VAR = 1.0
_PARAM_DIST = 1.0
_PARAM_REG = 0.001
_LOSS_WEIGHT = 1.0
_L = 50

_C = 8192


def _dl_kernel(pred_ref, sem_ref, inst_ref, out_ref,
               seg_ref, cnt_ref, mu_ref, lv_ref, *, n, klast):
    p = pl.program_id(1)
    k = pl.program_id(2)
    f = pred_ref.shape[1]

    x = pred_ref[0]
    sem = sem_ref[0]
    inst = inst_ref[0]

    col = jax.lax.broadcasted_iota(jnp.int32, (1, _C), 1)
    valid = (k * _C + col) < n
    keep = jnp.logical_and(sem != _IGNORE, valid)
    w = keep.astype(jnp.float32)
    label = jnp.where(sem == 1, 0, inst)
    label = jnp.where(keep, label, 0)

    x = jnp.where(valid, x, 0.0)

    rows = jax.lax.broadcasted_iota(jnp.int32, (_L, _C), 0)
    hit = jnp.logical_and(label == rows, keep)
    onehot = jnp.where(hit, jnp.float32(1), jnp.float32(0)).astype(jnp.bfloat16)

    @pl.when(jnp.logical_and(p == 0, k == 0))
    def _init_pass0():
        seg_ref[...] = jnp.zeros_like(seg_ref)
        cnt_ref[...] = jnp.zeros_like(cnt_ref)

    @pl.when(p == 0)
    def _pass0():
        xh = x.astype(jnp.bfloat16)
        xl = (x - xh.astype(jnp.float32)).astype(jnp.bfloat16)
        ones = jnp.ones((1, _C), jnp.bfloat16)
        aug = jnp.concatenate([xh, xl, ones], axis=0)
        res = jax.lax.dot_general(
            onehot, aug, (((1,), (1,)), ((), ())),
            preferred_element_type=jnp.float32)
        seg_ref[...] += res[:, :f] + res[:, f:2 * f]
        cnt_ref[...] += res[:, 2 * f:2 * f + 1]

    @pl.when(jnp.logical_and(p == 1, k == 0))
    def _init_pass1():
        cnt = cnt_ref[...]
        mu_ref[...] = seg_ref[...] / (cnt + 1e-8)
        lv_ref[...] = jnp.zeros_like(lv_ref)

    @pl.when(p == 1)
    def _pass1():
        mu_t = mu_ref[...].T
        mh = mu_t.astype(jnp.bfloat16)
        ml = (mu_t - mh.astype(jnp.float32)).astype(jnp.bfloat16)
        stacked = jnp.concatenate([mh, ml], axis=0)
        gath = jax.lax.dot_general(
            stacked, onehot, (((1,), (0,)), ((), ())),
            preferred_element_type=jnp.float32)
        mu_exp = gath[:f, :] + gath[f:, :]
        dist = jnp.sum(jnp.abs(x - mu_exp), axis=0, keepdims=True)
        dist = jnp.square(jnp.maximum(dist - _DELTA_V, 0.0)) * w
        dh = dist.astype(jnp.bfloat16)
        dl = (dist - dh.astype(jnp.float32)).astype(jnp.bfloat16)
        dd = jnp.concatenate([dh, dl], axis=0)
        lv_ref[...] += jax.lax.dot_general(
            onehot, dd, (((1,), (1,)), ((), ())),
            preferred_element_type=jnp.float32)

    @pl.when(jnp.logical_and(p == 1, k == klast))
    def _finish():
        cnt = cnt_ref[...]
        mu = mu_ref[...]
        present = (cnt > 0.0).astype(jnp.float32)
        ninst = jnp.sum(present)
        lvv = jnp.sum(lv_ref[...], axis=1, keepdims=True)
        l_var = jnp.sum(present * lvv / (cnt + 1e-8)) / ninst

        mu_t = mu.T
        norm = jnp.zeros((_L, _L), jnp.float32)
        for j in range(f):
            norm = norm + jnp.abs(mu[:, j:j + 1] - mu_t[j:j + 1, :])
        hinge = jnp.square(jnp.maximum(2.0 * _DELTA_D - norm, 0.0))
        ii = jax.lax.broadcasted_iota(jnp.int32, (_L, _L), 0)
        jj = jax.lax.broadcasted_iota(jnp.int32, (_L, _L), 1)
        pair_mask = present * present.T * jnp.where(ii == jj, 0.0, 1.0)
        l_dist = jnp.sum(pair_mask * hinge) / jnp.sum(pair_mask)

        l_reg = jnp.sum(present * jnp.sum(jnp.abs(mu), axis=1, keepdims=True)) / ninst

        r = jax.lax.broadcasted_iota(jnp.int32, (8, 128), 0)
        c = jax.lax.broadcasted_iota(jnp.int32, (8, 128), 1)
        first = (c == 0)
        packed = (jnp.where(jnp.logical_and(r == 0, first), l_var, 0.0)
                  + jnp.where(jnp.logical_and(r == 1, first), l_dist, 0.0)
                  + jnp.where(jnp.logical_and(r == 2, first), l_reg, 0.0))
        out_ref[0] = packed


def kernel(embedding_logits, semantic_labels, instance_labels, feature_dim):
    b, f, n = embedding_logits.shape
    k = -(-n // _C)
    sem3 = semantic_labels.reshape(b, 1, n)
    inst3 = instance_labels.reshape(b, 1, n)
    out = pl.pallas_call(
        functools.partial(_dl_kernel, n=n, klast=k - 1),
        grid=(b, 2, k),
        in_specs=[
            pl.BlockSpec((1, f, _C), lambda bi, pi, ki: (bi, 0, ki)),
            pl.BlockSpec((1, 1, _C), lambda bi, pi, ki: (bi, 0, ki)),
            pl.BlockSpec((1, 1, _C), lambda bi, pi, ki: (bi, 0, ki)),
        ],
        out_specs=pl.BlockSpec((1, 8, 128), lambda bi, pi, ki: (bi, 0, 0)),
        out_shape=jax.ShapeDtypeStruct((b, 8, 128), jnp.float32),
        scratch_shapes=[
            pltpu.VMEM((_L, f), jnp.float32),
            pltpu.VMEM((_L, 1), jnp.float32),
            pltpu.VMEM((_L, f), jnp.float32),
            pltpu.VMEM((_L, 2), jnp.float32),
        ],
    )(embedding_logits, sem3, inst3)
    l_var = _PARAM_VAR * out[:, 0, 0]
    l_dist = _PARAM_DIST * out[:, 1, 0]
    l_reg = _PARAM_REG * out[:, 2, 0]
    loss = _LOSS_WEIGHT * (l_var + l_dist + l_reg)
    scale = (jnp.asarray(feature_dim) // f).astype(jnp.float32)
    return (jnp.mean(loss) * scale, jnp.mean(l_var) * scale,
            jnp.mean(l_dist) * scale, jnp.mean(l_reg) * scale)


# speedup vs baseline: 22.0655x; 1.2212x over previous
import functools

import jax
import jax.numpy as jnp
from jax.experimental import pallas as pl
from jax.experimental.pallas import tpu as pltpu

_IGNORE = 0
_DELTA_V = 0.5
_DELTA_D = 1.5
_PARAM_VAR = 1.0
_PARAM_DIST = 1.0
_PARAM_REG = 0.001
_LOSS_WEIGHT = 1.0
_L = 50

_C = 8192


def _dl_kernel(pred_ref, sem_ref, inst_ref, out_ref,
               seg_ref, cnt_ref, mu_ref, lv_ref, xbuf_ref, ohbuf_ref,
               *, n, klast):
    p = pl.program_id(1)
    k = pl.program_id(2)
    f = pred_ref.shape[1]

    @pl.when(jnp.logical_and(p == 0, k == 0))
    def _init_pass0():
        seg_ref[...] = jnp.zeros_like(seg_ref)
        cnt_ref[...] = jnp.zeros_like(cnt_ref)

    @pl.when(p == 0)
    def _pass0():
        sem = sem_ref[0]
        inst = inst_ref[0]
        col = jax.lax.broadcasted_iota(jnp.int32, (1, _C), 1)
        valid = (k * _C + col) < n
        keep = jnp.logical_and(sem != _IGNORE, valid)
        label = jnp.where(sem == 1, 0, inst)
        rows = jax.lax.broadcasted_iota(jnp.int32, (_L, _C), 0)
        hit = jnp.logical_and(label == rows, keep)
        onehot = jnp.where(hit, jnp.float32(1), jnp.float32(0)
                           ).astype(jnp.bfloat16)

        xs = jnp.where(valid, pred_ref[0], 0.0)
        xbuf_ref[:, pl.ds(k * _C, _C)] = xs
        ohbuf_ref[:, pl.ds(k * _C, _C)] = onehot

        xh = xs.astype(jnp.bfloat16)
        xl = (xs - xh.astype(jnp.float32)).astype(jnp.bfloat16)
        ones = jnp.ones((1, _C), jnp.bfloat16)
        xlo = jnp.concatenate([xl, ones], axis=0)
        dn = (((1,), (1,)), ((), ()))
        hi = jax.lax.dot_general(onehot, xh, dn,
                                 preferred_element_type=jnp.float32)
        lo = jax.lax.dot_general(onehot, xlo, dn,
                                 preferred_element_type=jnp.float32)
        seg_ref[...] += hi + lo[:, :f]
        cnt_ref[...] += lo[:, f:f + 1]

    @pl.when(jnp.logical_and(p == 1, k == 0))
    def _init_pass1():
        cnt = cnt_ref[...]
        mu_ref[...] = seg_ref[...] / (cnt + 1e-8)
        lv_ref[...] = jnp.zeros_like(lv_ref)

    @pl.when(p == 1)
    def _pass1():
        xs = xbuf_ref[:, pl.ds(k * _C, _C)]
        onehot = ohbuf_ref[:, pl.ds(k * _C, _C)]
        mu_t = mu_ref[...].T
        mh = mu_t.astype(jnp.bfloat16)
        ml = (mu_t - mh.astype(jnp.float32)).astype(jnp.bfloat16)
        stacked = jnp.concatenate([mh, ml], axis=0)
        gath = jax.lax.dot_general(
            stacked, onehot, (((1,), (0,)), ((), ())),
            preferred_element_type=jnp.float32)
        mu_exp = gath[:f, :] + gath[f:, :]
        dist = jnp.sum(jnp.abs(xs - mu_exp), axis=0, keepdims=True)
        dist = jnp.square(jnp.maximum(dist - _DELTA_V, 0.0))
        dh = dist.astype(jnp.bfloat16)
        dl = (dist - dh.astype(jnp.float32)).astype(jnp.bfloat16)
        dd = jnp.concatenate([dh, dl], axis=0)
        lv_ref[...] += jax.lax.dot_general(
            onehot, dd, (((1,), (1,)), ((), ())),
            preferred_element_type=jnp.float32)

    @pl.when(jnp.logical_and(p == 1, k == klast))
    def _finish():
        cnt = cnt_ref[...]
        mu = mu_ref[...]
        present = (cnt > 0.0).astype(jnp.float32)
        ninst = jnp.sum(present)
        lvv = jnp.sum(lv_ref[...], axis=1, keepdims=True)
        l_var = jnp.sum(present * lvv / (cnt + 1e-8)) / ninst

        mu_t = mu.T
        norm = jnp.zeros((_L, _L), jnp.float32)
        for j in range(f):
            norm = norm + jnp.abs(mu[:, j:j + 1] - mu_t[j:j + 1, :])
        hinge = jnp.square(jnp.maximum(2.0 * _DELTA_D - norm, 0.0))
        ii = jax.lax.broadcasted_iota(jnp.int32, (_L, _L), 0)
        jj = jax.lax.broadcasted_iota(jnp.int32, (_L, _L), 1)
        pair_mask = present * present.T * jnp.where(ii == jj, 0.0, 1.0)
        l_dist = jnp.sum(pair_mask * hinge) / jnp.sum(pair_mask)

        l_reg = jnp.sum(present * jnp.sum(jnp.abs(mu), axis=1, keepdims=True)) / ninst

        r = jax.lax.broadcasted_iota(jnp.int32, (8, 128), 0)
        c = jax.lax.broadcasted_iota(jnp.int32, (8, 128), 1)
        first = (c == 0)
        packed = (jnp.where(jnp.logical_and(r == 0, first), l_var, 0.0)
                  + jnp.where(jnp.logical_and(r == 1, first), l_dist, 0.0)
                  + jnp.where(jnp.logical_and(r == 2, first), l_reg, 0.0))
        out_ref[0] = packed


def kernel(embedding_logits, semantic_labels, instance_labels, feature_dim):
    b, f, n = embedding_logits.shape
    k = -(-n // _C)
    sem3 = semantic_labels.reshape(b, 1, n)
    inst3 = instance_labels.reshape(b, 1, n)
    out = pl.pallas_call(
        functools.partial(_dl_kernel, n=n, klast=k - 1),
        grid=(b, 2, k),
        in_specs=[
            pl.BlockSpec((1, f, _C), lambda bi, pi, ki: (bi, 0, ki * (1 - pi))),
            pl.BlockSpec((1, 1, _C), lambda bi, pi, ki: (bi, 0, ki * (1 - pi))),
            pl.BlockSpec((1, 1, _C), lambda bi, pi, ki: (bi, 0, ki * (1 - pi))),
        ],
        out_specs=pl.BlockSpec((1, 8, 128), lambda bi, pi, ki: (bi, 0, 0)),
        out_shape=jax.ShapeDtypeStruct((b, 8, 128), jnp.float32),
        scratch_shapes=[
            pltpu.VMEM((_L, f), jnp.float32),
            pltpu.VMEM((_L, 1), jnp.float32),
            pltpu.VMEM((_L, f), jnp.float32),
            pltpu.VMEM((_L, 2), jnp.float32),
            pltpu.VMEM((f, k * _C), jnp.float32),
            pltpu.VMEM((_L, k * _C), jnp.bfloat16),
        ],
    )(embedding_logits, sem3, inst3)
    l_var = _PARAM_VAR * out[:, 0, 0]
    l_dist = _PARAM_DIST * out[:, 1, 0]
    l_reg = _PARAM_REG * out[:, 2, 0]
    loss = _LOSS_WEIGHT * (l_var + l_dist + l_reg)
    scale = (jnp.asarray(feature_dim) // f).astype(jnp.float32)
    return (jnp.mean(loss) * scale, jnp.mean(l_var) * scale,
            jnp.mean(l_dist) * scale, jnp.mean(l_reg) * scale)


# speedup vs baseline: 22.8662x; 1.0363x over previous
import functools

import jax
import jax.numpy as jnp
from jax.experimental import pallas as pl
from jax.experimental.pallas import tpu as pltpu

_IGNORE = 0
_DELTA_V = 0.5
_DELTA_D = 1.5
_PARAM_VAR = 1.0
_PARAM_DIST = 1.0
_PARAM_REG = 0.001
_LOSS_WEIGHT = 1.0
_L = 50

_C = 8192


def _dl_kernel(pred_ref, sem_ref, inst_ref, out_ref,
               seg_ref, cnt_ref, mu_ref, lv_ref, xbuf_ref, ohbuf_ref,
               *, n, klast):
    p = pl.program_id(1)
    k = pl.program_id(2)
    f = pred_ref.shape[1]

    @pl.when(jnp.logical_and(p == 0, k == 0))
    def _init_pass0():
        seg_ref[...] = jnp.zeros_like(seg_ref)
        cnt_ref[...] = jnp.zeros_like(cnt_ref)

    @pl.when(p == 0)
    def _pass0():
        sem = sem_ref[0]
        inst = inst_ref[0]
        col = jax.lax.broadcasted_iota(jnp.int32, (1, _C), 1)
        valid = (k * _C + col) < n
        keep = jnp.logical_and(sem != _IGNORE, valid)
        label = jnp.where(keep, jnp.where(sem == 1, 0, inst), -1)
        lab16 = label.astype(jnp.int16)
        rows16 = jax.lax.broadcasted_iota(jnp.int16, (_L, _C), 0)
        onehot = jnp.where(lab16 == rows16,
                           jnp.bfloat16(1), jnp.bfloat16(0))

        xs = jnp.where(valid, pred_ref[0], 0.0)
        xbuf_ref[:, pl.ds(k * _C, _C)] = xs
        ohbuf_ref[:, pl.ds(k * _C, _C)] = onehot

        xh = xs.astype(jnp.bfloat16)
        xl = (xs - xh.astype(jnp.float32)).astype(jnp.bfloat16)
        ones = jnp.ones((1, _C), jnp.bfloat16)
        xlo = jnp.concatenate([xl, ones], axis=0)
        dn = (((1,), (1,)), ((), ()))
        hi = jax.lax.dot_general(onehot, xh, dn,
                                 preferred_element_type=jnp.float32)
        lo = jax.lax.dot_general(onehot, xlo, dn,
                                 preferred_element_type=jnp.float32)
        seg_ref[...] += hi + lo[:, :f]
        cnt_ref[...] += lo[:, f:f + 1]

    @pl.when(jnp.logical_and(p == 1, k == 0))
    def _init_pass1():
        cnt = cnt_ref[...]
        mu_ref[...] = seg_ref[...] / (cnt + 1e-8)
        lv_ref[...] = jnp.zeros_like(lv_ref)

    @pl.when(p == 1)
    def _pass1():
        xs = xbuf_ref[:, pl.ds(k * _C, _C)]
        onehot = ohbuf_ref[:, pl.ds(k * _C, _C)]
        mu_t = mu_ref[...].T
        mh = mu_t.astype(jnp.bfloat16)
        ml = (mu_t - mh.astype(jnp.float32)).astype(jnp.bfloat16)
        stacked = jnp.concatenate([mh, ml], axis=0)
        gath = jax.lax.dot_general(
            stacked, onehot, (((1,), (0,)), ((), ())),
            preferred_element_type=jnp.float32)
        mu_exp = gath[:f, :] + gath[f:, :]
        dist = jnp.sum(jnp.abs(xs - mu_exp), axis=0, keepdims=True)
        dist = jnp.square(jnp.maximum(dist - _DELTA_V, 0.0))
        dh = dist.astype(jnp.bfloat16)
        dl = (dist - dh.astype(jnp.float32)).astype(jnp.bfloat16)
        dd = jnp.concatenate([dh, dl], axis=0)
        lv_ref[...] += jax.lax.dot_general(
            onehot, dd, (((1,), (1,)), ((), ())),
            preferred_element_type=jnp.float32)

    @pl.when(jnp.logical_and(p == 1, k == klast))
    def _finish():
        cnt = cnt_ref[...]
        mu = mu_ref[...]
        present = (cnt > 0.0).astype(jnp.float32)
        ninst = jnp.sum(present)
        lvv = jnp.sum(lv_ref[...], axis=1, keepdims=True)
        l_var = jnp.sum(present * lvv / (cnt + 1e-8)) / ninst

        mu_t = mu.T
        norm = jnp.zeros((_L, _L), jnp.float32)
        for j in range(f):
            norm = norm + jnp.abs(mu[:, j:j + 1] - mu_t[j:j + 1, :])
        hinge = jnp.square(jnp.maximum(2.0 * _DELTA_D - norm, 0.0))
        ii = jax.lax.broadcasted_iota(jnp.int32, (_L, _L), 0)
        jj = jax.lax.broadcasted_iota(jnp.int32, (_L, _L), 1)
        pair_mask = present * present.T * jnp.where(ii == jj, 0.0, 1.0)
        l_dist = jnp.sum(pair_mask * hinge) / jnp.sum(pair_mask)

        l_reg = jnp.sum(present * jnp.sum(jnp.abs(mu), axis=1, keepdims=True)) / ninst

        r = jax.lax.broadcasted_iota(jnp.int32, (8, 128), 0)
        c = jax.lax.broadcasted_iota(jnp.int32, (8, 128), 1)
        first = (c == 0)
        packed = (jnp.where(jnp.logical_and(r == 0, first), l_var, 0.0)
                  + jnp.where(jnp.logical_and(r == 1, first), l_dist, 0.0)
                  + jnp.where(jnp.logical_and(r == 2, first), l_reg, 0.0))
        out_ref[0] = packed


def kernel(embedding_logits, semantic_labels, instance_labels, feature_dim):
    b, f, n = embedding_logits.shape
    k = -(-n // _C)
    sem3 = semantic_labels.reshape(b, 1, n)
    inst3 = instance_labels.reshape(b, 1, n)
    out = pl.pallas_call(
        functools.partial(_dl_kernel, n=n, klast=k - 1),
        grid=(b, 2, k),
        in_specs=[
            pl.BlockSpec((1, f, _C), lambda bi, pi, ki: (bi, 0, ki * (1 - pi))),
            pl.BlockSpec((1, 1, _C), lambda bi, pi, ki: (bi, 0, ki * (1 - pi))),
            pl.BlockSpec((1, 1, _C), lambda bi, pi, ki: (bi, 0, ki * (1 - pi))),
        ],
        out_specs=pl.BlockSpec((1, 8, 128), lambda bi, pi, ki: (bi, 0, 0)),
        out_shape=jax.ShapeDtypeStruct((b, 8, 128), jnp.float32),
        scratch_shapes=[
            pltpu.VMEM((_L, f), jnp.float32),
            pltpu.VMEM((_L, 1), jnp.float32),
            pltpu.VMEM((_L, f), jnp.float32),
            pltpu.VMEM((_L, 2), jnp.float32),
            pltpu.VMEM((f, k * _C), jnp.float32),
            pltpu.VMEM((_L, k * _C), jnp.bfloat16),
        ],
    )(embedding_logits, sem3, inst3)
    l_var = _PARAM_VAR * out[:, 0, 0]
    l_dist = _PARAM_DIST * out[:, 1, 0]
    l_reg = _PARAM_REG * out[:, 2, 0]
    loss = _LOSS_WEIGHT * (l_var + l_dist + l_reg)
    scale = (jnp.asarray(feature_dim) // f).astype(jnp.float32)
    return (jnp.mean(loss) * scale, jnp.mean(l_var) * scale,
            jnp.mean(l_dist) * scale, jnp.mean(l_reg) * scale)


# speedup vs baseline: 27.5226x; 1.2036x over previous
import functools

import jax
import jax.numpy as jnp
from jax.experimental import pallas as pl
from jax.experimental.pallas import tpu as pltpu

_IGNORE = 0
_DELTA_V = 0.5
_DELTA_D = 1.5
_PARAM_VAR = 1.0
_PARAM_DIST = 1.0
_PARAM_REG = 0.001
_LOSS_WEIGHT = 1.0
_L = 50

_C = 12800


def _dl_kernel(pred_ref, sem_ref, inst_ref, out_ref,
               seg_ref, cnt_ref, mu_ref, lv_ref, xbuf_ref, ohbuf_ref,
               *, n, klast):
    p = pl.program_id(1)
    k = pl.program_id(2)
    f = pred_ref.shape[1]

    @pl.when(jnp.logical_and(p == 0, k == 0))
    def _init_pass0():
        seg_ref[...] = jnp.zeros_like(seg_ref)
        cnt_ref[...] = jnp.zeros_like(cnt_ref)

    @pl.when(p == 0)
    def _pass0():
        sem = sem_ref[0]
        inst = inst_ref[0]
        col = jax.lax.broadcasted_iota(jnp.int32, (1, _C), 1)
        valid = (k * _C + col) < n
        keep = jnp.logical_and(sem != _IGNORE, valid)
        label = jnp.where(keep, jnp.where(sem == 1, 0, inst), -1)
        lab16 = label.astype(jnp.int16)
        rows16 = jax.lax.broadcasted_iota(jnp.int16, (_L, _C), 0)
        onehot = jnp.where(lab16 == rows16,
                           jnp.bfloat16(1), jnp.bfloat16(0))

        xs = jnp.where(valid, pred_ref[0], 0.0)
        xbuf_ref[:, pl.ds(k * _C, _C)] = xs
        ohbuf_ref[:, pl.ds(k * _C, _C)] = onehot

        xh = xs.astype(jnp.bfloat16)
        xl = (xs - xh.astype(jnp.float32)).astype(jnp.bfloat16)
        ones = jnp.ones((1, _C), jnp.bfloat16)
        xlo = jnp.concatenate([xl, ones], axis=0)
        dn = (((1,), (1,)), ((), ()))
        hi = jax.lax.dot_general(onehot, xh, dn,
                                 preferred_element_type=jnp.float32)
        lo = jax.lax.dot_general(onehot, xlo, dn,
                                 preferred_element_type=jnp.float32)
        seg_ref[...] += hi + lo[:, :f]
        cnt_ref[...] += lo[:, f:f + 1]

    @pl.when(jnp.logical_and(p == 1, k == 0))
    def _init_pass1():
        cnt = cnt_ref[...]
        mu_ref[...] = seg_ref[...] / (cnt + 1e-8)
        lv_ref[...] = jnp.zeros_like(lv_ref)

    @pl.when(p == 1)
    def _pass1():
        xs = xbuf_ref[:, pl.ds(k * _C, _C)]
        onehot = ohbuf_ref[:, pl.ds(k * _C, _C)]
        mu_t = mu_ref[...].T
        mh = mu_t.astype(jnp.bfloat16)
        ml = (mu_t - mh.astype(jnp.float32)).astype(jnp.bfloat16)
        stacked = jnp.concatenate([mh, ml], axis=0)
        gath = jax.lax.dot_general(
            stacked, onehot, (((1,), (0,)), ((), ())),
            preferred_element_type=jnp.float32)
        mu_exp = gath[:f, :] + gath[f:, :]
        dist = jnp.sum(jnp.abs(xs - mu_exp), axis=0, keepdims=True)
        dist = jnp.square(jnp.maximum(dist - _DELTA_V, 0.0))
        dh = dist.astype(jnp.bfloat16)
        dl = (dist - dh.astype(jnp.float32)).astype(jnp.bfloat16)
        dd = jnp.concatenate([dh, dl], axis=0)
        lv_ref[...] += jax.lax.dot_general(
            onehot, dd, (((1,), (1,)), ((), ())),
            preferred_element_type=jnp.float32)

    @pl.when(jnp.logical_and(p == 1, k == klast))
    def _finish():
        cnt = cnt_ref[...]
        mu = mu_ref[...]
        present = (cnt > 0.0).astype(jnp.float32)
        ninst = jnp.sum(present)
        lvv = jnp.sum(lv_ref[...], axis=1, keepdims=True)
        l_var = jnp.sum(present * lvv / (cnt + 1e-8)) / ninst

        mu_t = mu.T
        norm = jnp.zeros((_L, _L), jnp.float32)
        for j in range(f):
            norm = norm + jnp.abs(mu[:, j:j + 1] - mu_t[j:j + 1, :])
        hinge = jnp.square(jnp.maximum(2.0 * _DELTA_D - norm, 0.0))
        ii = jax.lax.broadcasted_iota(jnp.int32, (_L, _L), 0)
        jj = jax.lax.broadcasted_iota(jnp.int32, (_L, _L), 1)
        pair_mask = present * present.T * jnp.where(ii == jj, 0.0, 1.0)
        l_dist = jnp.sum(pair_mask * hinge) / jnp.sum(pair_mask)

        l_reg = jnp.sum(present * jnp.sum(jnp.abs(mu), axis=1, keepdims=True)) / ninst

        r = jax.lax.broadcasted_iota(jnp.int32, (8, 128), 0)
        c = jax.lax.broadcasted_iota(jnp.int32, (8, 128), 1)
        first = (c == 0)
        packed = (jnp.where(jnp.logical_and(r == 0, first), l_var, 0.0)
                  + jnp.where(jnp.logical_and(r == 1, first), l_dist, 0.0)
                  + jnp.where(jnp.logical_and(r == 2, first), l_reg, 0.0))
        out_ref[0] = packed


def kernel(embedding_logits, semantic_labels, instance_labels, feature_dim):
    b, f, n = embedding_logits.shape
    k = -(-n // _C)
    sem3 = semantic_labels.reshape(b, 1, n)
    inst3 = instance_labels.reshape(b, 1, n)
    out = pl.pallas_call(
        functools.partial(_dl_kernel, n=n, klast=k - 1),
        grid=(b, 2, k),
        in_specs=[
            pl.BlockSpec((1, f, _C), lambda bi, pi, ki: (bi, 0, ki * (1 - pi))),
            pl.BlockSpec((1, 1, _C), lambda bi, pi, ki: (bi, 0, ki * (1 - pi))),
            pl.BlockSpec((1, 1, _C), lambda bi, pi, ki: (bi, 0, ki * (1 - pi))),
        ],
        out_specs=pl.BlockSpec((1, 8, 128), lambda bi, pi, ki: (bi, 0, 0)),
        out_shape=jax.ShapeDtypeStruct((b, 8, 128), jnp.float32),
        scratch_shapes=[
            pltpu.VMEM((_L, f), jnp.float32),
            pltpu.VMEM((_L, 1), jnp.float32),
            pltpu.VMEM((_L, f), jnp.float32),
            pltpu.VMEM((_L, 2), jnp.float32),
            pltpu.VMEM((f, k * _C), jnp.float32),
            pltpu.VMEM((_L, k * _C), jnp.bfloat16),
        ],
    )(embedding_logits, sem3, inst3)
    l_var = _PARAM_VAR * out[:, 0, 0]
    l_dist = _PARAM_DIST * out[:, 1, 0]
    l_reg = _PARAM_REG * out[:, 2, 0]
    loss = _LOSS_WEIGHT * (l_var + l_dist + l_reg)
    scale = (jnp.asarray(feature_dim) // f).astype(jnp.float32)
    return (jnp.mean(loss) * scale, jnp.mean(l_var) * scale,
            jnp.mean(l_dist) * scale, jnp.mean(l_reg) * scale)


# speedup vs baseline: 30.5531x; 1.1101x over previous
import functools

import jax
import jax.numpy as jnp
from jax.experimental import pallas as pl
from jax.experimental.pallas import tpu as pltpu

_IGNORE = 0
_DELTA_V = 0.5
_DELTA_D = 1.5
_PARAM_VAR = 1.0
_PARAM_DIST = 1.0
_PARAM_REG = 0.001
_LOSS_WEIGHT = 1.0
_L = 50

_C = 25600


def _dl_kernel(pred_ref, sem_ref, inst_ref, out_ref,
               seg_ref, cnt_ref, mu_ref, lv_ref, xbuf_ref, ohbuf_ref,
               *, n, klast):
    p = pl.program_id(1)
    k = pl.program_id(2)
    f = pred_ref.shape[1]

    @pl.when(jnp.logical_and(p == 0, k == 0))
    def _init_pass0():
        seg_ref[...] = jnp.zeros_like(seg_ref)
        cnt_ref[...] = jnp.zeros_like(cnt_ref)

    @pl.when(p == 0)
    def _pass0():
        sem = sem_ref[0]
        inst = inst_ref[0]
        col = jax.lax.broadcasted_iota(jnp.int32, (1, _C), 1)
        valid = (k * _C + col) < n
        keep = jnp.logical_and(sem != _IGNORE, valid)
        label = jnp.where(keep, jnp.where(sem == 1, 0, inst), -1)
        lab16 = label.astype(jnp.int16)
        rows16 = jax.lax.broadcasted_iota(jnp.int16, (_L, _C), 0)
        onehot = jnp.where(lab16 == rows16,
                           jnp.bfloat16(1), jnp.bfloat16(0))

        xs = jnp.where(valid, pred_ref[0], 0.0)
        xbuf_ref[:, pl.ds(k * _C, _C)] = xs
        ohbuf_ref[:, pl.ds(k * _C, _C)] = onehot

        xh = xs.astype(jnp.bfloat16)
        xl = (xs - xh.astype(jnp.float32)).astype(jnp.bfloat16)
        ones = jnp.ones((1, _C), jnp.bfloat16)
        xlo = jnp.concatenate([xl, ones], axis=0)
        dn = (((1,), (1,)), ((), ()))
        hi = jax.lax.dot_general(onehot, xh, dn,
                                 preferred_element_type=jnp.float32)
        lo = jax.lax.dot_general(onehot, xlo, dn,
                                 preferred_element_type=jnp.float32)
        seg_ref[...] += hi + lo[:, :f]
        cnt_ref[...] += lo[:, f:f + 1]

    @pl.when(jnp.logical_and(p == 1, k == 0))
    def _init_pass1():
        cnt = cnt_ref[...]
        mu_ref[...] = seg_ref[...] / (cnt + 1e-8)
        lv_ref[...] = jnp.zeros_like(lv_ref)

    @pl.when(p == 1)
    def _pass1():
        xs = xbuf_ref[:, pl.ds(k * _C, _C)]
        onehot = ohbuf_ref[:, pl.ds(k * _C, _C)]
        mu_t = mu_ref[...].T
        mh = mu_t.astype(jnp.bfloat16)
        ml = (mu_t - mh.astype(jnp.float32)).astype(jnp.bfloat16)
        stacked = jnp.concatenate([mh, ml], axis=0)
        gath = jax.lax.dot_general(
            stacked, onehot, (((1,), (0,)), ((), ())),
            preferred_element_type=jnp.float32)
        mu_exp = gath[:f, :] + gath[f:, :]
        dist = jnp.sum(jnp.abs(xs - mu_exp), axis=0, keepdims=True)
        dist = jnp.square(jnp.maximum(dist - _DELTA_V, 0.0))
        dh = dist.astype(jnp.bfloat16)
        dl = (dist - dh.astype(jnp.float32)).astype(jnp.bfloat16)
        dd = jnp.concatenate([dh, dl], axis=0)
        lv_ref[...] += jax.lax.dot_general(
            onehot, dd, (((1,), (1,)), ((), ())),
            preferred_element_type=jnp.float32)

    @pl.when(jnp.logical_and(p == 1, k == klast))
    def _finish():
        cnt = cnt_ref[...]
        mu = mu_ref[...]
        present = (cnt > 0.0).astype(jnp.float32)
        ninst = jnp.sum(present)
        lvv = jnp.sum(lv_ref[...], axis=1, keepdims=True)
        l_var = jnp.sum(present * lvv / (cnt + 1e-8)) / ninst

        mu_t = mu.T
        norm = jnp.zeros((_L, _L), jnp.float32)
        for j in range(f):
            norm = norm + jnp.abs(mu[:, j:j + 1] - mu_t[j:j + 1, :])
        hinge = jnp.square(jnp.maximum(2.0 * _DELTA_D - norm, 0.0))
        ii = jax.lax.broadcasted_iota(jnp.int32, (_L, _L), 0)
        jj = jax.lax.broadcasted_iota(jnp.int32, (_L, _L), 1)
        pair_mask = present * present.T * jnp.where(ii == jj, 0.0, 1.0)
        l_dist = jnp.sum(pair_mask * hinge) / jnp.sum(pair_mask)

        l_reg = jnp.sum(present * jnp.sum(jnp.abs(mu), axis=1, keepdims=True)) / ninst

        r = jax.lax.broadcasted_iota(jnp.int32, (8, 128), 0)
        c = jax.lax.broadcasted_iota(jnp.int32, (8, 128), 1)
        first = (c == 0)
        packed = (jnp.where(jnp.logical_and(r == 0, first), l_var, 0.0)
                  + jnp.where(jnp.logical_and(r == 1, first), l_dist, 0.0)
                  + jnp.where(jnp.logical_and(r == 2, first), l_reg, 0.0))
        out_ref[0] = packed


def kernel(embedding_logits, semantic_labels, instance_labels, feature_dim):
    b, f, n = embedding_logits.shape
    k = -(-n // _C)
    sem3 = semantic_labels.reshape(b, 1, n)
    inst3 = instance_labels.reshape(b, 1, n)
    out = pl.pallas_call(
        functools.partial(_dl_kernel, n=n, klast=k - 1),
        grid=(b, 2, k),
        in_specs=[
            pl.BlockSpec((1, f, _C), lambda bi, pi, ki: (bi, 0, ki * (1 - pi))),
            pl.BlockSpec((1, 1, _C), lambda bi, pi, ki: (bi, 0, ki * (1 - pi))),
            pl.BlockSpec((1, 1, _C), lambda bi, pi, ki: (bi, 0, ki * (1 - pi))),
        ],
        out_specs=pl.BlockSpec((1, 8, 128), lambda bi, pi, ki: (bi, 0, 0)),
        out_shape=jax.ShapeDtypeStruct((b, 8, 128), jnp.float32),
        scratch_shapes=[
            pltpu.VMEM((_L, f), jnp.float32),
            pltpu.VMEM((_L, 1), jnp.float32),
            pltpu.VMEM((_L, f), jnp.float32),
            pltpu.VMEM((_L, 2), jnp.float32),
            pltpu.VMEM((f, k * _C), jnp.float32),
            pltpu.VMEM((_L, k * _C), jnp.bfloat16),
        ],
    )(embedding_logits, sem3, inst3)
    l_var = _PARAM_VAR * out[:, 0, 0]
    l_dist = _PARAM_DIST * out[:, 1, 0]
    l_reg = _PARAM_REG * out[:, 2, 0]
    loss = _LOSS_WEIGHT * (l_var + l_dist + l_reg)
    scale = (jnp.asarray(feature_dim) // f).astype(jnp.float32)
    return (jnp.mean(loss) * scale, jnp.mean(l_var) * scale,
            jnp.mean(l_dist) * scale, jnp.mean(l_reg) * scale)


# speedup vs baseline: 31.9184x; 1.0447x over previous
import functools

import jax
import jax.numpy as jnp
from jax.experimental import pallas as pl
from jax.experimental.pallas import tpu as pltpu

_IGNORE = 0
_DELTA_V = 0.5
_DELTA_D = 1.5
_PARAM_VAR = 1.0
_PARAM_DIST = 1.0
_PARAM_REG = 0.001
_LOSS_WEIGHT = 1.0
_L = 50

_C = 51200


def _dl_kernel(pred_ref, sem_ref, inst_ref, out_ref,
               seg_ref, cnt_ref, mu_ref, lv_ref, xbuf_ref, ohbuf_ref,
               *, n, klast):
    p = pl.program_id(1)
    k = pl.program_id(2)
    f = pred_ref.shape[1]

    @pl.when(jnp.logical_and(p == 0, k == 0))
    def _init_pass0():
        seg_ref[...] = jnp.zeros_like(seg_ref)
        cnt_ref[...] = jnp.zeros_like(cnt_ref)

    @pl.when(p == 0)
    def _pass0():
        sem = sem_ref[0]
        inst = inst_ref[0]
        col = jax.lax.broadcasted_iota(jnp.int32, (1, _C), 1)
        valid = (k * _C + col) < n
        keep = jnp.logical_and(sem != _IGNORE, valid)
        label = jnp.where(keep, jnp.where(sem == 1, 0, inst), -1)
        lab16 = label.astype(jnp.int16)
        rows16 = jax.lax.broadcasted_iota(jnp.int16, (_L, _C), 0)
        onehot = jnp.where(lab16 == rows16,
                           jnp.bfloat16(1), jnp.bfloat16(0))

        xs = jnp.where(valid, pred_ref[0], 0.0)
        xbuf_ref[:, pl.ds(k * _C, _C)] = xs
        ohbuf_ref[:, pl.ds(k * _C, _C)] = onehot

        xh = xs.astype(jnp.bfloat16)
        xl = (xs - xh.astype(jnp.float32)).astype(jnp.bfloat16)
        ones = jnp.ones((1, _C), jnp.bfloat16)
        xlo = jnp.concatenate([xl, ones], axis=0)
        dn = (((1,), (1,)), ((), ()))
        hi = jax.lax.dot_general(onehot, xh, dn,
                                 preferred_element_type=jnp.float32)
        lo = jax.lax.dot_general(onehot, xlo, dn,
                                 preferred_element_type=jnp.float32)
        seg_ref[...] += hi + lo[:, :f]
        cnt_ref[...] += lo[:, f:f + 1]

    @pl.when(jnp.logical_and(p == 1, k == 0))
    def _init_pass1():
        cnt = cnt_ref[...]
        mu_ref[...] = seg_ref[...] / (cnt + 1e-8)
        lv_ref[...] = jnp.zeros_like(lv_ref)

    @pl.when(p == 1)
    def _pass1():
        xs = xbuf_ref[:, pl.ds(k * _C, _C)]
        onehot = ohbuf_ref[:, pl.ds(k * _C, _C)]
        mu_t = mu_ref[...].T
        mh = mu_t.astype(jnp.bfloat16)
        ml = (mu_t - mh.astype(jnp.float32)).astype(jnp.bfloat16)
        stacked = jnp.concatenate([mh, ml], axis=0)
        gath = jax.lax.dot_general(
            stacked, onehot, (((1,), (0,)), ((), ())),
            preferred_element_type=jnp.float32)
        mu_exp = gath[:f, :] + gath[f:, :]
        dist = jnp.sum(jnp.abs(xs - mu_exp), axis=0, keepdims=True)
        dist = jnp.square(jnp.maximum(dist - _DELTA_V, 0.0))
        dh = dist.astype(jnp.bfloat16)
        dl = (dist - dh.astype(jnp.float32)).astype(jnp.bfloat16)
        dd = jnp.concatenate([dh, dl], axis=0)
        lv_ref[...] += jax.lax.dot_general(
            onehot, dd, (((1,), (1,)), ((), ())),
            preferred_element_type=jnp.float32)

    @pl.when(jnp.logical_and(p == 1, k == klast))
    def _finish():
        cnt = cnt_ref[...]
        mu = mu_ref[...]
        present = (cnt > 0.0).astype(jnp.float32)
        ninst = jnp.sum(present)
        lvv = jnp.sum(lv_ref[...], axis=1, keepdims=True)
        l_var = jnp.sum(present * lvv / (cnt + 1e-8)) / ninst

        mu_t = mu.T
        norm = jnp.zeros((_L, _L), jnp.float32)
        for j in range(f):
            norm = norm + jnp.abs(mu[:, j:j + 1] - mu_t[j:j + 1, :])
        hinge = jnp.square(jnp.maximum(2.0 * _DELTA_D - norm, 0.0))
        ii = jax.lax.broadcasted_iota(jnp.int32, (_L, _L), 0)
        jj = jax.lax.broadcasted_iota(jnp.int32, (_L, _L), 1)
        pair_mask = present * present.T * jnp.where(ii == jj, 0.0, 1.0)
        l_dist = jnp.sum(pair_mask * hinge) / jnp.sum(pair_mask)

        l_reg = jnp.sum(present * jnp.sum(jnp.abs(mu), axis=1, keepdims=True)) / ninst

        r = jax.lax.broadcasted_iota(jnp.int32, (8, 128), 0)
        c = jax.lax.broadcasted_iota(jnp.int32, (8, 128), 1)
        first = (c == 0)
        packed = (jnp.where(jnp.logical_and(r == 0, first), l_var, 0.0)
                  + jnp.where(jnp.logical_and(r == 1, first), l_dist, 0.0)
                  + jnp.where(jnp.logical_and(r == 2, first), l_reg, 0.0))
        out_ref[0] = packed


def kernel(embedding_logits, semantic_labels, instance_labels, feature_dim):
    b, f, n = embedding_logits.shape
    k = -(-n // _C)
    sem3 = semantic_labels.reshape(b, 1, n)
    inst3 = instance_labels.reshape(b, 1, n)
    out = pl.pallas_call(
        functools.partial(_dl_kernel, n=n, klast=k - 1),
        grid=(b, 2, k),
        in_specs=[
            pl.BlockSpec((1, f, _C), lambda bi, pi, ki: (bi, 0, ki * (1 - pi))),
            pl.BlockSpec((1, 1, _C), lambda bi, pi, ki: (bi, 0, ki * (1 - pi))),
            pl.BlockSpec((1, 1, _C), lambda bi, pi, ki: (bi, 0, ki * (1 - pi))),
        ],
        out_specs=pl.BlockSpec((1, 8, 128), lambda bi, pi, ki: (bi, 0, 0)),
        out_shape=jax.ShapeDtypeStruct((b, 8, 128), jnp.float32),
        scratch_shapes=[
            pltpu.VMEM((_L, f), jnp.float32),
            pltpu.VMEM((_L, 1), jnp.float32),
            pltpu.VMEM((_L, f), jnp.float32),
            pltpu.VMEM((_L, 2), jnp.float32),
            pltpu.VMEM((f, k * _C), jnp.float32),
            pltpu.VMEM((_L, k * _C), jnp.bfloat16),
        ],
    )(embedding_logits, sem3, inst3)
    l_var = _PARAM_VAR * out[:, 0, 0]
    l_dist = _PARAM_DIST * out[:, 1, 0]
    l_reg = _PARAM_REG * out[:, 2, 0]
    loss = _LOSS_WEIGHT * (l_var + l_dist + l_reg)
    scale = (jnp.asarray(feature_dim) // f).astype(jnp.float32)
    return (jnp.mean(loss) * scale, jnp.mean(l_var) * scale,
            jnp.mean(l_dist) * scale, jnp.mean(l_reg) * scale)
